# Initial kernel scaffold; baseline (speedup 1.0000x reference)
import jax
import jax.numpy as jnp
from jax.experimental import pallas as pl


def kernel(x, edge_index, W1, b1, W2, b2):
    raise NotImplementedError("write your pallas kernel here")


# speedup vs baseline: 20.3579x; 20.3579x over previous
import functools

import jax
import jax.numpy as jnp
from jax import lax
from jax.experimental import pallas as pl
from jax.experimental.pallas import tpu as pltpu
from jax.experimental.pallas import tpu_sc as plsc

_NC = 2
_NS = 16
_NW = _NC * _NS
_CHUNK = 128
_F = 8


def _round_up(v, m):
    return (v + m - 1) // m * m


def _make_deg_kernel(n_pad, e_pad):
    nb = e_pad // (_CHUNK * _NW)
    rpt = n_pad // _NS
    mesh = plsc.VectorSubcoreMesh(core_axis_name="c", subcore_axis_name="s",
                                  num_cores=_NC, num_subcores=_NS)

    @functools.partial(
        pl.kernel,
        out_type=jax.ShapeDtypeStruct((_NC * n_pad, _F), jnp.float32),
        mesh=mesh,
        scratch_types=[
            pltpu.VMEM((_CHUNK,), jnp.int32),
            pltpu.VMEM((_CHUNK, _F), jnp.float32),
            pltpu.VMEM_SHARED((n_pad, _F), jnp.float32),
        ],
        compiler_params=pltpu.CompilerParams(use_tc_tiling_on_sc=False),
    )
    def deg_kernel(dst_hbm, zeros_hbm, ones_hbm, out_hbm, didx, obuf, acc):
        c = lax.axis_index("c")
        s = lax.axis_index("s")
        wid = c * _NS + s
        pltpu.sync_copy(ones_hbm, obuf)
        pltpu.sync_copy(zeros_hbm, acc.at[pl.ds(s * rpt, rpt)])
        plsc.subcore_barrier()

        def chunk(b, _):
            pltpu.sync_copy(dst_hbm.at[wid * nb + b], didx)
            pltpu.sync_copy(obuf, acc.at[didx], add=True)
            return _

        lax.fori_loop(0, nb, chunk, None)
        plsc.subcore_barrier()
        pltpu.sync_copy(acc.at[pl.ds(s * rpt, rpt)],
                        out_hbm.at[pl.ds((c * _NS + s) * rpt, rpt)])

    return deg_kernel


def _make_edge_kernel(n_pad, e_pad):
    nb = e_pad // (_CHUNK * _NW)
    rpt = n_pad // _NS
    mesh = plsc.VectorSubcoreMesh(core_axis_name="c", subcore_axis_name="s",
                                  num_cores=_NC, num_subcores=_NS)

    @functools.partial(
        pl.kernel,
        out_type=jax.ShapeDtypeStruct((_NC * n_pad, _F), jnp.float32),
        mesh=mesh,
        scratch_types=[
            pltpu.VMEM((_CHUNK,), jnp.int32),
            pltpu.VMEM((_CHUNK,), jnp.int32),
            pltpu.VMEM((_CHUNK, _F), jnp.float32),
            pltpu.VMEM_SHARED((n_pad, _F), jnp.float32),
            pltpu.SemaphoreType.DMA,
        ],
        compiler_params=pltpu.CompilerParams(use_tc_tiling_on_sc=False),
    )
    def edge_kernel(vals_hbm, src_hbm, dst_hbm, zeros_hbm, out_hbm,
                    sidx, didx, grows, acc, sem):
        c = lax.axis_index("c")
        s = lax.axis_index("s")
        wid = c * _NS + s
        pltpu.sync_copy(zeros_hbm, acc.at[pl.ds(s * rpt, rpt)])
        plsc.subcore_barrier()

        def chunk(b, _):
            pltpu.sync_copy(src_hbm.at[wid * nb + b], sidx)
            pltpu.sync_copy(dst_hbm.at[wid * nb + b], didx)
            pltpu.async_copy(vals_hbm.at[sidx], grows, sem).wait()
            pltpu.sync_copy(grows, acc.at[didx], add=True)
            return _

        lax.fori_loop(0, nb, chunk, None)
        plsc.subcore_barrier()
        pltpu.sync_copy(acc.at[pl.ds(s * rpt, rpt)],
                        out_hbm.at[pl.ds((c * _NS + s) * rpt, rpt)])

    return edge_kernel


_BLK = 2048


def _tc_norm(d0_ref, d1_ref, x_ref, dinv_ref, ys_ref):
    deg = d0_ref[...] + d1_ref[...] + 1.0
    dinv = lax.rsqrt(deg)
    dinv_ref[...] = dinv
    f_in = x_ref.shape[1]
    ys_ref[...] = jnp.concatenate(
        [x_ref[...] * dinv,
         jnp.zeros((x_ref.shape[0], _F - f_in), jnp.float32)], axis=1)


def _tc_mid(a0_ref, a1_ref, ys_ref, dinv_ref, w1_ref, b1_ref, w2_ref, qs_ref):
    agg = (a0_ref[...] + a1_ref[...] + ys_ref[...]) * dinv_ref[...]
    h = jnp.maximum(
        jnp.dot(agg, w1_ref[...], preferred_element_type=jnp.float32)
        + b1_ref[...], 0.0)
    q = jnp.dot(h, w2_ref[...], preferred_element_type=jnp.float32)
    qs_ref[...] = jnp.concatenate(
        [q * dinv_ref[...],
         jnp.zeros((q.shape[0], _F - q.shape[1]), jnp.float32)], axis=1)


def _tc_final(c0_ref, c1_ref, qs_ref, dinv_ref, b2_ref, out_ref):
    out_ref[...] = ((c0_ref[...] + c1_ref[...] + qs_ref[...])
                    * dinv_ref[...] + b2_ref[...])


def kernel(x, edge_index, W1, b1, W2, b2):
    n = x.shape[0]
    e = edge_index.shape[1]
    f_in = x.shape[1]
    f_out = W2.shape[1]
    n_pad = _round_up(n, 2048)
    e_pad = _round_up(e, _CHUNK * _NW)
    rpt = n_pad // _NS

    src = edge_index[0].astype(jnp.int32)
    dst = edge_index[1].astype(jnp.int32)
    src_p = jnp.concatenate(
        [src, jnp.zeros((e_pad - e,), jnp.int32)]).reshape(e_pad // _CHUNK, _CHUNK)
    dst_p = jnp.concatenate(
        [dst, jnp.full((e_pad - e,), n, jnp.int32)]).reshape(e_pad // _CHUNK, _CHUNK)
    x_p = jnp.concatenate([x, jnp.zeros((n_pad - n, f_in), x.dtype)])

    zeros8 = jnp.zeros((rpt, _F), jnp.float32)
    ones8 = jnp.concatenate(
        [jnp.ones((_CHUNK, 1), jnp.float32),
         jnp.zeros((_CHUNK, _F - 1), jnp.float32)], axis=1)

    deg_parts = _make_deg_kernel(n_pad, e_pad)(dst_p, zeros8, ones8)
    d0 = deg_parts[:n_pad, :1]
    d1 = deg_parts[n_pad:, :1]

    grid = (n_pad // _BLK,)

    def _blk(f):
        return pl.BlockSpec((_BLK, f), lambda i: (i, 0))

    def _full(shape):
        return pl.BlockSpec(shape, lambda i: (0, 0))

    dinv, ys8 = pl.pallas_call(
        _tc_norm,
        grid=grid,
        in_specs=[_blk(1), _blk(1), _blk(f_in)],
        out_specs=[_blk(1), _blk(_F)],
        out_shape=[jax.ShapeDtypeStruct((n_pad, 1), jnp.float32),
                   jax.ShapeDtypeStruct((n_pad, _F), jnp.float32)],
    )(d0, d1, x_p)

    a_parts = _make_edge_kernel(n_pad, e_pad)(ys8, src_p, dst_p, zeros8)
    a0 = a_parts[:n_pad, :f_in]
    a1 = a_parts[n_pad:, :f_in]
    ys = ys8[:, :f_in]

    qs8 = pl.pallas_call(
        _tc_mid,
        grid=grid,
        in_specs=[_blk(f_in), _blk(f_in), _blk(f_in), _blk(1),
                  _full(W1.shape), _full((1, b1.shape[0])), _full(W2.shape)],
        out_specs=_blk(_F),
        out_shape=jax.ShapeDtypeStruct((n_pad, _F), jnp.float32),
    )(a0, a1, ys, dinv, W1, b1.reshape(1, -1), W2)

    c_parts = _make_edge_kernel(n_pad, e_pad)(qs8, src_p, dst_p, zeros8)
    c0 = c_parts[:n_pad, :f_out]
    c1 = c_parts[n_pad:, :f_out]
    qs = qs8[:, :f_out]

    out = pl.pallas_call(
        _tc_final,
        grid=grid,
        in_specs=[_blk(f_out), _blk(f_out), _blk(f_out), _blk(1),
                  _full((1, b2.shape[0]))],
        out_specs=_blk(f_out),
        out_shape=jax.ShapeDtypeStruct((n_pad, f_out), jnp.float32),
    )(c0, c1, qs, dinv, b2.reshape(1, -1))
    return out[:n]


# speedup vs baseline: 56.1752x; 2.7594x over previous
import functools

import jax
import jax.numpy as jnp
from jax import lax
from jax.experimental import pallas as pl
from jax.experimental.pallas import tpu as pltpu
from jax.experimental.pallas import tpu_sc as plsc

_NC = 2
_NS = 16
_NW = _NC * _NS
_CHUNK = 128
_F = 8
_INNER = 8


def _round_up(v, m):
    return (v + m - 1) // m * m


def _make_deg_kernel(n_pad, e_pad):
    nb = e_pad // (_CHUNK * _NW)
    ng = nb // _INNER
    rpt = n_pad // _NS
    mesh = plsc.VectorSubcoreMesh(core_axis_name="c", subcore_axis_name="s",
                                  num_cores=_NC, num_subcores=_NS)

    @functools.partial(
        pl.kernel,
        out_type=jax.ShapeDtypeStruct((_NC * n_pad, _F), jnp.float32),
        mesh=mesh,
        scratch_types=[
            pltpu.VMEM((2, _INNER, _CHUNK), jnp.int32),
            pltpu.VMEM((_CHUNK, _F), jnp.float32),
            pltpu.VMEM_SHARED((n_pad, _F), jnp.float32),
            pltpu.SemaphoreType.DMA,
            pltpu.SemaphoreType.DMA,
        ],
        compiler_params=pltpu.CompilerParams(use_tc_tiling_on_sc=False),
    )
    def deg_kernel(dst_hbm, zeros_hbm, ones_hbm, out_hbm, didx, obuf, acc,
                   isem, ssem):
        c = lax.axis_index("c")
        s = lax.axis_index("s")
        wid = c * _NS + s
        pltpu.sync_copy(ones_hbm, obuf)
        pltpu.sync_copy(zeros_hbm, acc.at[pl.ds(s * rpt, rpt)])
        plsc.subcore_barrier()
        base = wid * nb
        pltpu.async_copy(dst_hbm.at[pl.ds(base, _INNER)], didx.at[0], isem)

        def group(g, _):
            p = lax.rem(g, 2)

            @pl.when(g >= 1)
            def _drain_prev():
                for j in range(_INNER):
                    pltpu.make_async_copy(
                        obuf, acc.at[didx.at[1 - p, j]], ssem).wait()

            @pl.when(g + 1 < ng)
            def _prefetch():
                pltpu.async_copy(
                    dst_hbm.at[pl.ds(base + (g + 1) * _INNER, _INNER)],
                    didx.at[1 - p], isem)

            pltpu.make_async_copy(
                dst_hbm.at[pl.ds(base, _INNER)], didx.at[p], isem).wait()
            for j in range(_INNER):
                pltpu.async_copy(obuf, acc.at[didx.at[p, j]], ssem, add=True)
            return _

        lax.fori_loop(0, ng, group, None)
        pf = lax.rem(ng - 1, 2)
        for j in range(_INNER):
            pltpu.make_async_copy(obuf, acc.at[didx.at[pf, j]], ssem).wait()
        plsc.subcore_barrier()
        pltpu.sync_copy(acc.at[pl.ds(s * rpt, rpt)],
                        out_hbm.at[pl.ds((c * _NS + s) * rpt, rpt)])

    return deg_kernel


def _make_edge_kernel(n_pad, e_pad):
    nb = e_pad // (_CHUNK * _NW)
    ng = nb // _INNER
    rpt = n_pad // _NS
    mesh = plsc.VectorSubcoreMesh(core_axis_name="c", subcore_axis_name="s",
                                  num_cores=_NC, num_subcores=_NS)

    @functools.partial(
        pl.kernel,
        out_type=jax.ShapeDtypeStruct((_NC * n_pad, _F), jnp.float32),
        mesh=mesh,
        scratch_types=[
            pltpu.VMEM((2, _INNER, _CHUNK), jnp.int32),
            pltpu.VMEM((2, _INNER, _CHUNK), jnp.int32),
            pltpu.VMEM((2, _INNER, _CHUNK, _F), jnp.float32),
            pltpu.VMEM_SHARED((n_pad, _F), jnp.float32),
            pltpu.SemaphoreType.DMA,
            pltpu.SemaphoreType.DMA,
            pltpu.SemaphoreType.DMA,
        ],
        compiler_params=pltpu.CompilerParams(use_tc_tiling_on_sc=False),
    )
    def edge_kernel(vals_hbm, src_hbm, dst_hbm, zeros_hbm, out_hbm,
                    sidx, didx, grows, acc, isem, gsem, ssem):
        c = lax.axis_index("c")
        s = lax.axis_index("s")
        wid = c * _NS + s
        pltpu.sync_copy(zeros_hbm, acc.at[pl.ds(s * rpt, rpt)])
        plsc.subcore_barrier()
        base = wid * nb
        pltpu.async_copy(src_hbm.at[pl.ds(base, _INNER)], sidx.at[0], isem)
        pltpu.async_copy(dst_hbm.at[pl.ds(base, _INNER)], didx.at[0], isem)

        def group(g, _):
            p = lax.rem(g, 2)

            @pl.when(g >= 1)
            def _drain_prev():
                for j in range(_INNER):
                    pltpu.make_async_copy(
                        grows.at[1 - p, j], acc.at[didx.at[1 - p, j]],
                        ssem).wait()

            @pl.when(g + 1 < ng)
            def _prefetch():
                off = base + (g + 1) * _INNER
                pltpu.async_copy(src_hbm.at[pl.ds(off, _INNER)],
                                 sidx.at[1 - p], isem)
                pltpu.async_copy(dst_hbm.at[pl.ds(off, _INNER)],
                                 didx.at[1 - p], isem)

            pltpu.make_async_copy(
                src_hbm.at[pl.ds(base, _INNER)], sidx.at[p], isem).wait()
            pltpu.make_async_copy(
                dst_hbm.at[pl.ds(base, _INNER)], didx.at[p], isem).wait()
            for j in range(_INNER):
                pltpu.async_copy(vals_hbm.at[sidx.at[p, j]],
                                 grows.at[p, j], gsem)
            for j in range(_INNER):
                pltpu.make_async_copy(vals_hbm.at[sidx.at[p, j]],
                                      grows.at[p, j], gsem).wait()
            for j in range(_INNER):
                pltpu.async_copy(grows.at[p, j], acc.at[didx.at[p, j]],
                                 ssem, add=True)
            return _

        lax.fori_loop(0, ng, group, None)
        pf = lax.rem(ng - 1, 2)
        for j in range(_INNER):
            pltpu.make_async_copy(
                grows.at[pf, j], acc.at[didx.at[pf, j]], ssem).wait()
        plsc.subcore_barrier()
        pltpu.sync_copy(acc.at[pl.ds(s * rpt, rpt)],
                        out_hbm.at[pl.ds((c * _NS + s) * rpt, rpt)])

    return edge_kernel


_BLK = 2048


def _tc_norm(d0_ref, d1_ref, x_ref, dinv_ref, ys_ref):
    deg = d0_ref[...] + d1_ref[...] + 1.0
    dinv = lax.rsqrt(deg)
    dinv_ref[...] = dinv
    f_in = x_ref.shape[1]
    ys_ref[...] = jnp.concatenate(
        [x_ref[...] * dinv,
         jnp.zeros((x_ref.shape[0], _F - f_in), jnp.float32)], axis=1)


def _tc_mid(a0_ref, a1_ref, ys_ref, dinv_ref, w1_ref, b1_ref, w2_ref, qs_ref):
    agg = (a0_ref[...] + a1_ref[...] + ys_ref[...]) * dinv_ref[...]
    h = jnp.maximum(
        jnp.dot(agg, w1_ref[...], preferred_element_type=jnp.float32)
        + b1_ref[...], 0.0)
    q = jnp.dot(h, w2_ref[...], preferred_element_type=jnp.float32)
    qs_ref[...] = jnp.concatenate(
        [q * dinv_ref[...],
         jnp.zeros((q.shape[0], _F - q.shape[1]), jnp.float32)], axis=1)


def _tc_final(c0_ref, c1_ref, qs_ref, dinv_ref, b2_ref, out_ref):
    out_ref[...] = ((c0_ref[...] + c1_ref[...] + qs_ref[...])
                    * dinv_ref[...] + b2_ref[...])


def kernel(x, edge_index, W1, b1, W2, b2):
    n = x.shape[0]
    e = edge_index.shape[1]
    f_in = x.shape[1]
    f_out = W2.shape[1]
    n_pad = _round_up(n, 2048)
    e_pad = _round_up(e, _CHUNK * _INNER * _NW)
    rpt = n_pad // _NS

    src = edge_index[0].astype(jnp.int32)
    dst = edge_index[1].astype(jnp.int32)
    src_p = jnp.concatenate(
        [src, jnp.zeros((e_pad - e,), jnp.int32)]).reshape(e_pad // _CHUNK, _CHUNK)
    dst_p = jnp.concatenate(
        [dst, jnp.full((e_pad - e,), n, jnp.int32)]).reshape(e_pad // _CHUNK, _CHUNK)
    x_p = jnp.concatenate([x, jnp.zeros((n_pad - n, f_in), x.dtype)])

    zeros8 = jnp.zeros((rpt, _F), jnp.float32)
    ones8 = jnp.concatenate(
        [jnp.ones((_CHUNK, 1), jnp.float32),
         jnp.zeros((_CHUNK, _F - 1), jnp.float32)], axis=1)

    deg_parts = _make_deg_kernel(n_pad, e_pad)(dst_p, zeros8, ones8)
    d0 = deg_parts[:n_pad, :1]
    d1 = deg_parts[n_pad:, :1]

    grid = (n_pad // _BLK,)

    def _blk(f):
        return pl.BlockSpec((_BLK, f), lambda i: (i, 0))

    def _full(shape):
        return pl.BlockSpec(shape, lambda i: (0, 0))

    dinv, ys8 = pl.pallas_call(
        _tc_norm,
        grid=grid,
        in_specs=[_blk(1), _blk(1), _blk(f_in)],
        out_specs=[_blk(1), _blk(_F)],
        out_shape=[jax.ShapeDtypeStruct((n_pad, 1), jnp.float32),
                   jax.ShapeDtypeStruct((n_pad, _F), jnp.float32)],
    )(d0, d1, x_p)

    a_parts = _make_edge_kernel(n_pad, e_pad)(ys8, src_p, dst_p, zeros8)
    a0 = a_parts[:n_pad, :f_in]
    a1 = a_parts[n_pad:, :f_in]
    ys = ys8[:, :f_in]

    qs8 = pl.pallas_call(
        _tc_mid,
        grid=grid,
        in_specs=[_blk(f_in), _blk(f_in), _blk(f_in), _blk(1),
                  _full(W1.shape), _full((1, b1.shape[0])), _full(W2.shape)],
        out_specs=_blk(_F),
        out_shape=jax.ShapeDtypeStruct((n_pad, _F), jnp.float32),
    )(a0, a1, ys, dinv, W1, b1.reshape(1, -1), W2)

    c_parts = _make_edge_kernel(n_pad, e_pad)(qs8, src_p, dst_p, zeros8)
    c0 = c_parts[:n_pad, :f_out]
    c1 = c_parts[n_pad:, :f_out]
    qs = qs8[:, :f_out]

    out = pl.pallas_call(
        _tc_final,
        grid=grid,
        in_specs=[_blk(f_out), _blk(f_out), _blk(f_out), _blk(1),
                  _full((1, b2.shape[0]))],
        out_specs=_blk(f_out),
        out_shape=jax.ShapeDtypeStruct((n_pad, f_out), jnp.float32),
    )(c0, c1, qs, dinv, b2.reshape(1, -1))
    return out[:n]


# speedup vs baseline: 67.9918x; 1.2104x over previous
import functools

import jax
import jax.numpy as jnp
from jax import lax
from jax.experimental import pallas as pl
from jax.experimental.pallas import tpu as pltpu
from jax.experimental.pallas import tpu_sc as plsc

_NC = 2
_NS = 16
_NW = _NC * _NS
_CHUNK = 128
_F = 8
_INNER = 8


def _round_up(v, m):
    return (v + m - 1) // m * m


def _make_deg_kernel(n_pad, e_pad):
    nb = e_pad // (_CHUNK * _NW)
    ng = nb // _INNER
    rpt = n_pad // _NS
    mesh = plsc.VectorSubcoreMesh(core_axis_name="c", subcore_axis_name="s",
                                  num_cores=_NC, num_subcores=_NS)

    @functools.partial(
        pl.kernel,
        out_type=jax.ShapeDtypeStruct((_NC * n_pad, _F), jnp.float32),
        mesh=mesh,
        scratch_types=[
            pltpu.VMEM((2, _INNER, _CHUNK), jnp.int32),
            pltpu.VMEM((_CHUNK, _F), jnp.float32),
            pltpu.VMEM_SHARED((n_pad, _F), jnp.float32),
            pltpu.SemaphoreType.DMA,
            pltpu.SemaphoreType.DMA,
        ],
        compiler_params=pltpu.CompilerParams(use_tc_tiling_on_sc=False),
    )
    def deg_kernel(dst_hbm, zeros_hbm, ones_hbm, out_hbm, didx, obuf, acc,
                   isem, ssem):
        c = lax.axis_index("c")
        s = lax.axis_index("s")
        wid = c * _NS + s
        pltpu.sync_copy(ones_hbm, obuf)
        pltpu.sync_copy(zeros_hbm, acc.at[pl.ds(s * rpt, rpt)])
        plsc.subcore_barrier()
        base = wid * nb
        pltpu.async_copy(dst_hbm.at[pl.ds(base, _INNER)], didx.at[0], isem)

        def group(g, _):
            p = lax.rem(g, 2)

            @pl.when(g >= 1)
            def _drain_prev():
                for j in range(_INNER):
                    pltpu.make_async_copy(
                        obuf, acc.at[didx.at[1 - p, j]], ssem).wait()

            @pl.when(g + 1 < ng)
            def _prefetch():
                pltpu.async_copy(
                    dst_hbm.at[pl.ds(base + (g + 1) * _INNER, _INNER)],
                    didx.at[1 - p], isem)

            pltpu.make_async_copy(
                dst_hbm.at[pl.ds(base, _INNER)], didx.at[p], isem).wait()
            for j in range(_INNER):
                pltpu.async_copy(obuf, acc.at[didx.at[p, j]], ssem, add=True)
            return _

        lax.fori_loop(0, ng, group, None)
        pf = lax.rem(ng - 1, 2)
        for j in range(_INNER):
            pltpu.make_async_copy(obuf, acc.at[didx.at[pf, j]], ssem).wait()
        plsc.subcore_barrier()
        pltpu.sync_copy(acc.at[pl.ds(s * rpt, rpt)],
                        out_hbm.at[pl.ds((c * _NS + s) * rpt, rpt)])

    return deg_kernel


def _make_edge_kernel(n_pad, e_pad):
    nb = e_pad // (_CHUNK * _NW)
    ng = nb // _INNER
    rpt = n_pad // _NS
    mesh = plsc.VectorSubcoreMesh(core_axis_name="c", subcore_axis_name="s",
                                  num_cores=_NC, num_subcores=_NS)

    @functools.partial(
        pl.kernel,
        out_type=jax.ShapeDtypeStruct((_NC * n_pad, _F), jnp.float32),
        mesh=mesh,
        scratch_types=[
            pltpu.VMEM((2, _INNER, _CHUNK), jnp.int32),
            pltpu.VMEM((2, _INNER, _CHUNK), jnp.int32),
            pltpu.VMEM((2, _INNER, _CHUNK, _F), jnp.float32),
            pltpu.VMEM_SHARED((n_pad, _F), jnp.float32),
            pltpu.SemaphoreType.DMA,
            pltpu.SemaphoreType.DMA,
            pltpu.SemaphoreType.DMA,
        ],
        compiler_params=pltpu.CompilerParams(use_tc_tiling_on_sc=False),
    )
    def edge_kernel(vals_hbm, src_hbm, dst_hbm, zeros_hbm, out_hbm,
                    sidx, didx, grows, acc, isem, gsem, ssem):
        c = lax.axis_index("c")
        s = lax.axis_index("s")
        wid = c * _NS + s
        pltpu.sync_copy(zeros_hbm, acc.at[pl.ds(s * rpt, rpt)])
        plsc.subcore_barrier()
        base = wid * nb
        pltpu.async_copy(src_hbm.at[pl.ds(base, _INNER)], sidx.at[0], isem)
        pltpu.async_copy(dst_hbm.at[pl.ds(base, _INNER)], didx.at[0], isem)

        def group(g, _):
            p = lax.rem(g, 2)

            @pl.when(g >= 1)
            def _drain_prev():
                for j in range(_INNER):
                    pltpu.make_async_copy(
                        grows.at[1 - p, j], acc.at[didx.at[1 - p, j]],
                        ssem).wait()

            @pl.when(g + 1 < ng)
            def _prefetch():
                off = base + (g + 1) * _INNER
                pltpu.async_copy(src_hbm.at[pl.ds(off, _INNER)],
                                 sidx.at[1 - p], isem)
                pltpu.async_copy(dst_hbm.at[pl.ds(off, _INNER)],
                                 didx.at[1 - p], isem)

            pltpu.make_async_copy(
                src_hbm.at[pl.ds(base, _INNER)], sidx.at[p], isem).wait()
            pltpu.make_async_copy(
                dst_hbm.at[pl.ds(base, _INNER)], didx.at[p], isem).wait()
            for j in range(_INNER):
                pltpu.async_copy(vals_hbm.at[sidx.at[p, j]],
                                 grows.at[p, j], gsem)
            for j in range(_INNER):
                pltpu.make_async_copy(vals_hbm.at[sidx.at[p, j]],
                                      grows.at[p, j], gsem).wait()
                pltpu.async_copy(grows.at[p, j], acc.at[didx.at[p, j]],
                                 ssem, add=True)
            return _

        lax.fori_loop(0, ng, group, None)
        pf = lax.rem(ng - 1, 2)
        for j in range(_INNER):
            pltpu.make_async_copy(
                grows.at[pf, j], acc.at[didx.at[pf, j]], ssem).wait()
        plsc.subcore_barrier()
        pltpu.sync_copy(acc.at[pl.ds(s * rpt, rpt)],
                        out_hbm.at[pl.ds((c * _NS + s) * rpt, rpt)])

    return edge_kernel


_BLK = 2048


def _tc_norm(d0_ref, d1_ref, x_ref, dinv_ref, ys_ref):
    deg = d0_ref[:, :1] + d1_ref[:, :1] + 1.0
    dinv = lax.rsqrt(deg)
    dinv_ref[...] = dinv
    f_in = x_ref.shape[1]
    ys_ref[...] = jnp.concatenate(
        [x_ref[...] * dinv,
         jnp.zeros((x_ref.shape[0], _F - f_in), jnp.float32)], axis=1)


def _tc_mid(a0_ref, a1_ref, ys_ref, dinv_ref, w1_ref, b1_ref, w2_ref, qs_ref):
    f_in = w1_ref.shape[0]
    agg = ((a0_ref[:, :f_in] + a1_ref[:, :f_in] + ys_ref[:, :f_in])
           * dinv_ref[...])
    h = jnp.maximum(
        jnp.dot(agg, w1_ref[...], preferred_element_type=jnp.float32)
        + b1_ref[...], 0.0)
    q = jnp.dot(h, w2_ref[...], preferred_element_type=jnp.float32)
    qs_ref[...] = jnp.concatenate(
        [q * dinv_ref[...],
         jnp.zeros((q.shape[0], _F - q.shape[1]), jnp.float32)], axis=1)


def _tc_final(c0_ref, c1_ref, qs_ref, dinv_ref, b2_ref, out_ref):
    f_out = b2_ref.shape[1]
    out_ref[...] = ((c0_ref[:, :f_out] + c1_ref[:, :f_out]
                     + qs_ref[:, :f_out]) * dinv_ref[...] + b2_ref[...])


def kernel(x, edge_index, W1, b1, W2, b2):
    n = x.shape[0]
    e = edge_index.shape[1]
    f_in = x.shape[1]
    f_out = W2.shape[1]
    n_pad = _round_up(n, 2048)
    e_pad = _round_up(e, _CHUNK * _INNER * _NW)
    rpt = n_pad // _NS

    src = edge_index[0].astype(jnp.int32)
    dst = edge_index[1].astype(jnp.int32)
    src_p = jnp.concatenate(
        [src, jnp.zeros((e_pad - e,), jnp.int32)]).reshape(e_pad // _CHUNK, _CHUNK)
    dst_p = jnp.concatenate(
        [dst, jnp.full((e_pad - e,), n, jnp.int32)]).reshape(e_pad // _CHUNK, _CHUNK)
    x_p = jnp.concatenate([x, jnp.zeros((n_pad - n, f_in), x.dtype)])

    zeros8 = jnp.zeros((rpt, _F), jnp.float32)
    ones8 = jnp.concatenate(
        [jnp.ones((_CHUNK, 1), jnp.float32),
         jnp.zeros((_CHUNK, _F - 1), jnp.float32)], axis=1)

    deg_parts = _make_deg_kernel(n_pad, e_pad)(dst_p, zeros8, ones8)

    grid = (n_pad // _BLK,)

    def _blk(f):
        return pl.BlockSpec((_BLK, f), lambda i: (i, 0))

    def _full(shape):
        return pl.BlockSpec(shape, lambda i: (0, 0))

    nblk = n_pad // _BLK

    def _blk2(off):
        return pl.BlockSpec((_BLK, _F), lambda i, off=off: (off + i, 0))

    dinv, ys8 = pl.pallas_call(
        _tc_norm,
        grid=grid,
        in_specs=[_blk2(0), _blk2(nblk), _blk(f_in)],
        out_specs=[_blk(1), _blk(_F)],
        out_shape=[jax.ShapeDtypeStruct((n_pad, 1), jnp.float32),
                   jax.ShapeDtypeStruct((n_pad, _F), jnp.float32)],
    )(deg_parts, deg_parts, x_p)

    a_parts = _make_edge_kernel(n_pad, e_pad)(ys8, src_p, dst_p, zeros8)

    qs8 = pl.pallas_call(
        _tc_mid,
        grid=grid,
        in_specs=[_blk2(0), _blk2(nblk), _blk(_F), _blk(1),
                  _full(W1.shape), _full((1, b1.shape[0])), _full(W2.shape)],
        out_specs=_blk(_F),
        out_shape=jax.ShapeDtypeStruct((n_pad, _F), jnp.float32),
    )(a_parts, a_parts, ys8, dinv, W1, b1.reshape(1, -1), W2)

    c_parts = _make_edge_kernel(n_pad, e_pad)(qs8, src_p, dst_p, zeros8)

    out = pl.pallas_call(
        _tc_final,
        grid=grid,
        in_specs=[_blk2(0), _blk2(nblk), _blk(_F), _blk(1),
                  _full((1, b2.shape[0]))],
        out_specs=_blk(f_out),
        out_shape=jax.ShapeDtypeStruct((n_pad, f_out), jnp.float32),
    )(c_parts, c_parts, qs8, dinv, b2.reshape(1, -1))
    return out[:n]


# speedup vs baseline: 107.8724x; 1.5866x over previous
import functools

import jax
import jax.numpy as jnp
from jax import lax
from jax.experimental import pallas as pl
from jax.experimental.pallas import tpu as pltpu
from jax.experimental.pallas import tpu_sc as plsc

_NC = 2
_NS = 16
_NW = _NC * _NS
_CHUNK = 128
_F = 8
_INNER = 8


def _round_up(v, m):
    return (v + m - 1) // m * m


def _make_deg_kernel(n_pad, e_pad):
    nb = e_pad // (_CHUNK * _NW)
    ng = nb // _INNER
    rpt = n_pad // _NS
    mesh = plsc.VectorSubcoreMesh(core_axis_name="c", subcore_axis_name="s",
                                  num_cores=_NC, num_subcores=_NS)

    @functools.partial(
        pl.kernel,
        out_type=jax.ShapeDtypeStruct((_NC * n_pad, _F), jnp.float32),
        mesh=mesh,
        scratch_types=[
            pltpu.VMEM((2, _INNER, _CHUNK), jnp.int32),
            pltpu.VMEM((_CHUNK, _F), jnp.float32),
            pltpu.VMEM_SHARED((n_pad, _F), jnp.float32),
            pltpu.SemaphoreType.DMA,
            pltpu.SemaphoreType.DMA,
        ],
        compiler_params=pltpu.CompilerParams(use_tc_tiling_on_sc=False),
    )
    def deg_kernel(dst_hbm, zeros_hbm, ones_hbm, out_hbm, didx, obuf, acc,
                   isem, ssem):
        c = lax.axis_index("c")
        s = lax.axis_index("s")
        wid = c * _NS + s
        pltpu.sync_copy(ones_hbm, obuf)
        pltpu.sync_copy(zeros_hbm, acc.at[pl.ds(s * rpt, rpt)])
        plsc.subcore_barrier()
        base = wid * nb
        pltpu.async_copy(dst_hbm.at[pl.ds(base, _INNER)], didx.at[0], isem)

        def group(g, _):
            p = lax.rem(g, 2)

            @pl.when(g >= 1)
            def _drain_prev():
                for j in range(_INNER):
                    pltpu.make_async_copy(
                        obuf, acc.at[didx.at[1 - p, j]], ssem).wait()

            @pl.when(g + 1 < ng)
            def _prefetch():
                pltpu.async_copy(
                    dst_hbm.at[pl.ds(base + (g + 1) * _INNER, _INNER)],
                    didx.at[1 - p], isem)

            pltpu.make_async_copy(
                dst_hbm.at[pl.ds(base, _INNER)], didx.at[p], isem).wait()
            for j in range(_INNER):
                pltpu.async_copy(obuf, acc.at[didx.at[p, j]], ssem, add=True)
            return _

        lax.fori_loop(0, ng, group, None)
        pf = lax.rem(ng - 1, 2)
        for j in range(_INNER):
            pltpu.make_async_copy(obuf, acc.at[didx.at[pf, j]], ssem).wait()
        plsc.subcore_barrier()
        pltpu.sync_copy(acc.at[pl.ds(s * rpt, rpt)],
                        out_hbm.at[pl.ds((c * _NS + s) * rpt, rpt)])

    return deg_kernel


def _make_edge_kernel(n_pad, e_pad):
    nb = e_pad // (_CHUNK * _NW)
    ng = nb // _INNER
    rpt = n_pad // _NS
    mesh = plsc.VectorSubcoreMesh(core_axis_name="c", subcore_axis_name="s",
                                  num_cores=_NC, num_subcores=_NS)

    @functools.partial(
        pl.kernel,
        out_type=jax.ShapeDtypeStruct((_NC * n_pad, _F), jnp.float32),
        mesh=mesh,
        scratch_types=[
            pltpu.VMEM((2, _INNER, _CHUNK), jnp.int32),
            pltpu.VMEM((2, _INNER, _CHUNK), jnp.int32),
            pltpu.VMEM((2, _INNER, _CHUNK, _F), jnp.float32),
            pltpu.VMEM_SHARED((n_pad, _F), jnp.float32),
            pltpu.SemaphoreType.DMA,
            pltpu.SemaphoreType.DMA,
            pltpu.SemaphoreType.DMA,
        ],
        compiler_params=pltpu.CompilerParams(use_tc_tiling_on_sc=False),
    )
    def edge_kernel(vals_hbm, src_hbm, dst_hbm, zeros_hbm, out_hbm,
                    sidx, didx, grows, acc, isem, gsem, ssem):
        c = lax.axis_index("c")
        s = lax.axis_index("s")
        wid = c * _NS + s
        pltpu.sync_copy(zeros_hbm, acc.at[pl.ds(s * rpt, rpt)])
        plsc.subcore_barrier()
        base = wid * nb
        pltpu.async_copy(src_hbm.at[pl.ds(base, _INNER)], sidx.at[0], isem)
        pltpu.async_copy(dst_hbm.at[pl.ds(base, _INNER)], didx.at[0], isem)

        def group(g, _):
            p = lax.rem(g, 2)

            @pl.when(g >= 1)
            def _drain_prev():
                for j in range(_INNER):
                    pltpu.make_async_copy(
                        grows.at[1 - p, j], acc.at[didx.at[1 - p, j]],
                        ssem).wait()

            @pl.when(g + 1 < ng)
            def _prefetch():
                off = base + (g + 1) * _INNER
                pltpu.async_copy(src_hbm.at[pl.ds(off, _INNER)],
                                 sidx.at[1 - p], isem)
                pltpu.async_copy(dst_hbm.at[pl.ds(off, _INNER)],
                                 didx.at[1 - p], isem)

            pltpu.make_async_copy(
                src_hbm.at[pl.ds(base, _INNER)], sidx.at[p], isem).wait()
            pltpu.make_async_copy(
                dst_hbm.at[pl.ds(base, _INNER)], didx.at[p], isem).wait()
            for j in range(_INNER):
                pltpu.async_copy(vals_hbm.at[sidx.at[p, j]],
                                 grows.at[p, j], gsem)
            for j in range(_INNER):
                pltpu.make_async_copy(vals_hbm.at[sidx.at[p, j]],
                                      grows.at[p, j], gsem).wait()
                pltpu.async_copy(grows.at[p, j], acc.at[didx.at[p, j]],
                                 ssem, add=True)
            return _

        lax.fori_loop(0, ng, group, None)
        pf = lax.rem(ng - 1, 2)
        for j in range(_INNER):
            pltpu.make_async_copy(
                grows.at[pf, j], acc.at[didx.at[pf, j]], ssem).wait()
        plsc.subcore_barrier()
        pltpu.sync_copy(acc.at[pl.ds(s * rpt, rpt)],
                        out_hbm.at[pl.ds((c * _NS + s) * rpt, rpt)])

    return edge_kernel


def _tc_norm(d0_ref, d1_ref, x8_ref, dinv_ref, ys_ref):
    deg = d0_ref[...] + d1_ref[...] + 1.0
    dinv = lax.rsqrt(deg)
    dinv_ref[...] = dinv
    ys_ref[...] = x8_ref[...] * dinv


def _tc_mid(a0_ref, a1_ref, ys_ref, dinv_ref, bd1_ref, b1t_ref, bd2_ref,
            qs_ref):
    agg = (a0_ref[...] + a1_ref[...] + ys_ref[...]) * dinv_ref[...]
    h = jnp.maximum(
        jnp.dot(agg, bd1_ref[...], preferred_element_type=jnp.float32)
        + b1t_ref[...], 0.0)
    q = jnp.dot(h, bd2_ref[...], preferred_element_type=jnp.float32)
    qs_ref[...] = q * dinv_ref[...]


def _tc_final(c0_ref, c1_ref, qs_ref, dinv_ref, b2t_ref, out_ref):
    out_ref[...] = ((c0_ref[...] + c1_ref[...] + qs_ref[...])
                    * dinv_ref[...] + b2t_ref[...])


def kernel(x, edge_index, W1, b1, W2, b2):
    n = x.shape[0]
    e = edge_index.shape[1]
    f_in = x.shape[1]
    f_hid = W1.shape[1]
    f_out = W2.shape[1]
    n_pad = _round_up(n, 2048)
    e_pad = _round_up(e, _CHUNK * _INNER * _NW)
    rpt = n_pad // _NS
    m = n_pad * _F // 128
    npl = 128 // _F

    src = edge_index[0].astype(jnp.int32)
    dst = edge_index[1].astype(jnp.int32)
    src_p = jnp.concatenate(
        [src, jnp.zeros((e_pad - e,), jnp.int32)]).reshape(e_pad // _CHUNK, _CHUNK)
    dst_p = jnp.concatenate(
        [dst, jnp.full((e_pad - e,), n, jnp.int32)]).reshape(e_pad // _CHUNK, _CHUNK)
    x8 = jnp.zeros((n_pad, _F), jnp.float32).at[:n, :f_in].set(x)

    zeros8 = jnp.zeros((rpt, _F), jnp.float32)
    ones8 = jnp.ones((_CHUNK, _F), jnp.float32)

    w1p = jnp.zeros((_F, f_hid), jnp.float32).at[:f_in].set(W1)
    w2p = jnp.zeros((f_hid, _F), jnp.float32).at[:, :f_out].set(W2)
    bd1 = jnp.kron(jnp.eye(npl, dtype=jnp.float32), w1p)
    bd2 = jnp.kron(jnp.eye(npl, dtype=jnp.float32), w2p)
    b1t = jnp.tile(b1, npl).reshape(1, npl * f_hid)
    b2t = jnp.tile(jnp.zeros((_F,), jnp.float32).at[:f_out].set(b2),
                   npl).reshape(1, 128)

    def _half(i):
        return pl.BlockSpec((m, 128), lambda g, i=i: (i, 0))

    def _fb(shape):
        return pl.BlockSpec(shape, lambda g: (0, 0))

    deg_parts = _make_deg_kernel(n_pad, e_pad)(dst_p, zeros8, ones8)
    degf = deg_parts.reshape(2 * m, 128)

    dinvf, ysf = pl.pallas_call(
        _tc_norm,
        grid=(1,),
        in_specs=[_half(0), _half(1), _fb((m, 128))],
        out_specs=[_fb((m, 128)), _fb((m, 128))],
        out_shape=[jax.ShapeDtypeStruct((m, 128), jnp.float32),
                   jax.ShapeDtypeStruct((m, 128), jnp.float32)],
    )(degf, degf, x8.reshape(m, 128))

    a_parts = _make_edge_kernel(n_pad, e_pad)(ysf.reshape(n_pad, _F),
                                              src_p, dst_p, zeros8)

    qsf = pl.pallas_call(
        _tc_mid,
        grid=(1,),
        in_specs=[_half(0), _half(1), _fb((m, 128)), _fb((m, 128)),
                  _fb(bd1.shape), _fb(b1t.shape), _fb(bd2.shape)],
        out_specs=_fb((m, 128)),
        out_shape=jax.ShapeDtypeStruct((m, 128), jnp.float32),
    )(a_parts.reshape(2 * m, 128), a_parts.reshape(2 * m, 128), ysf, dinvf,
      bd1, b1t, bd2)

    c_parts = _make_edge_kernel(n_pad, e_pad)(qsf.reshape(n_pad, _F),
                                              src_p, dst_p, zeros8)

    outf = pl.pallas_call(
        _tc_final,
        grid=(1,),
        in_specs=[_half(0), _half(1), _fb((m, 128)), _fb((m, 128)),
                  _fb(b2t.shape)],
        out_specs=_fb((m, 128)),
        out_shape=jax.ShapeDtypeStruct((m, 128), jnp.float32),
    )(c_parts.reshape(2 * m, 128), c_parts.reshape(2 * m, 128), qsf, dinvf,
      b2t)
    return outf.reshape(n_pad, _F)[:n, :f_out]


# speedup vs baseline: 118.5103x; 1.0986x over previous
import functools

import jax
import jax.numpy as jnp
from jax import lax
from jax.experimental import pallas as pl
from jax.experimental.pallas import tpu as pltpu
from jax.experimental.pallas import tpu_sc as plsc

_NC = 2
_NS = 16
_NW = _NC * _NS
_CHUNK = 128
_F = 8
_INNER = 16


def _round_up(v, m):
    return (v + m - 1) // m * m


def _make_deg_kernel(n_pad, e_pad):
    nb = e_pad // (_CHUNK * _NW)
    ng = nb // _INNER
    rpt = n_pad // _NS
    mesh = plsc.VectorSubcoreMesh(core_axis_name="c", subcore_axis_name="s",
                                  num_cores=_NC, num_subcores=_NS)

    @functools.partial(
        pl.kernel,
        out_type=jax.ShapeDtypeStruct((_NC * n_pad, _F), jnp.float32),
        mesh=mesh,
        scratch_types=[
            pltpu.VMEM((2, _INNER, _CHUNK), jnp.int32),
            pltpu.VMEM((_CHUNK, _F), jnp.float32),
            pltpu.VMEM_SHARED((n_pad, _F), jnp.float32),
            pltpu.SemaphoreType.DMA,
            pltpu.SemaphoreType.DMA,
        ],
        compiler_params=pltpu.CompilerParams(use_tc_tiling_on_sc=False),
    )
    def deg_kernel(dst_hbm, zeros_hbm, ones_hbm, out_hbm, didx, obuf, acc,
                   isem, ssem):
        c = lax.axis_index("c")
        s = lax.axis_index("s")
        wid = c * _NS + s
        pltpu.sync_copy(ones_hbm, obuf)
        pltpu.sync_copy(zeros_hbm, acc.at[pl.ds(s * rpt, rpt)])
        plsc.subcore_barrier()
        base = wid * nb
        pltpu.async_copy(dst_hbm.at[pl.ds(base, _INNER)], didx.at[0], isem)

        def group(g, _):
            p = lax.rem(g, 2)

            @pl.when(g >= 1)
            def _drain_prev():
                for j in range(_INNER):
                    pltpu.make_async_copy(
                        obuf, acc.at[didx.at[1 - p, j]], ssem).wait()

            @pl.when(g + 1 < ng)
            def _prefetch():
                pltpu.async_copy(
                    dst_hbm.at[pl.ds(base + (g + 1) * _INNER, _INNER)],
                    didx.at[1 - p], isem)

            pltpu.make_async_copy(
                dst_hbm.at[pl.ds(base, _INNER)], didx.at[p], isem).wait()
            for j in range(_INNER):
                pltpu.async_copy(obuf, acc.at[didx.at[p, j]], ssem, add=True)
            return _

        lax.fori_loop(0, ng, group, None)
        pf = lax.rem(ng - 1, 2)
        for j in range(_INNER):
            pltpu.make_async_copy(obuf, acc.at[didx.at[pf, j]], ssem).wait()
        plsc.subcore_barrier()
        pltpu.sync_copy(acc.at[pl.ds(s * rpt, rpt)],
                        out_hbm.at[pl.ds((c * _NS + s) * rpt, rpt)])

    return deg_kernel


def _make_edge_kernel(n_pad, e_pad):
    nb = e_pad // (_CHUNK * _NW)
    ng = nb // _INNER
    rpt = n_pad // _NS
    mesh = plsc.VectorSubcoreMesh(core_axis_name="c", subcore_axis_name="s",
                                  num_cores=_NC, num_subcores=_NS)

    @functools.partial(
        pl.kernel,
        out_type=jax.ShapeDtypeStruct((_NC * n_pad, _F), jnp.float32),
        mesh=mesh,
        scratch_types=[
            pltpu.VMEM((2, _INNER, _CHUNK), jnp.int32),
            pltpu.VMEM((2, _INNER, _CHUNK), jnp.int32),
            pltpu.VMEM((2, _INNER, _CHUNK, _F), jnp.float32),
            pltpu.VMEM_SHARED((n_pad, _F), jnp.float32),
            pltpu.SemaphoreType.DMA,
            pltpu.SemaphoreType.DMA,
            pltpu.SemaphoreType.DMA,
        ],
        compiler_params=pltpu.CompilerParams(use_tc_tiling_on_sc=False),
    )
    def edge_kernel(vals_hbm, src_hbm, dst_hbm, zeros_hbm, out_hbm,
                    sidx, didx, grows, acc, isem, gsem, ssem):
        c = lax.axis_index("c")
        s = lax.axis_index("s")
        wid = c * _NS + s
        pltpu.sync_copy(zeros_hbm, acc.at[pl.ds(s * rpt, rpt)])
        plsc.subcore_barrier()
        base = wid * nb
        pltpu.async_copy(src_hbm.at[pl.ds(base, _INNER)], sidx.at[0], isem)
        pltpu.async_copy(dst_hbm.at[pl.ds(base, _INNER)], didx.at[0], isem)

        def group(g, _):
            p = lax.rem(g, 2)

            @pl.when(g >= 1)
            def _drain_prev():
                for j in range(_INNER):
                    pltpu.make_async_copy(
                        grows.at[1 - p, j], acc.at[didx.at[1 - p, j]],
                        ssem).wait()

            @pl.when(g + 1 < ng)
            def _prefetch():
                off = base + (g + 1) * _INNER
                pltpu.async_copy(src_hbm.at[pl.ds(off, _INNER)],
                                 sidx.at[1 - p], isem)
                pltpu.async_copy(dst_hbm.at[pl.ds(off, _INNER)],
                                 didx.at[1 - p], isem)

            pltpu.make_async_copy(
                src_hbm.at[pl.ds(base, _INNER)], sidx.at[p], isem).wait()
            pltpu.make_async_copy(
                dst_hbm.at[pl.ds(base, _INNER)], didx.at[p], isem).wait()
            for j in range(_INNER):
                pltpu.async_copy(vals_hbm.at[sidx.at[p, j]],
                                 grows.at[p, j], gsem)
            for j in range(_INNER):
                pltpu.make_async_copy(vals_hbm.at[sidx.at[p, j]],
                                      grows.at[p, j], gsem).wait()
                pltpu.async_copy(grows.at[p, j], acc.at[didx.at[p, j]],
                                 ssem, add=True)
            return _

        lax.fori_loop(0, ng, group, None)
        pf = lax.rem(ng - 1, 2)
        for j in range(_INNER):
            pltpu.make_async_copy(
                grows.at[pf, j], acc.at[didx.at[pf, j]], ssem).wait()
        plsc.subcore_barrier()
        pltpu.sync_copy(acc.at[pl.ds(s * rpt, rpt)],
                        out_hbm.at[pl.ds((c * _NS + s) * rpt, rpt)])

    return edge_kernel


def _tc_norm(d0_ref, d1_ref, x8_ref, dinv_ref, ys_ref):
    deg = d0_ref[...] + d1_ref[...] + 1.0
    dinv = lax.rsqrt(deg)
    dinv_ref[...] = dinv
    ys_ref[...] = x8_ref[...] * dinv


def _tc_mid(a0_ref, a1_ref, ys_ref, dinv_ref, bd1_ref, b1t_ref, bd2_ref,
            qs_ref):
    agg = (a0_ref[...] + a1_ref[...] + ys_ref[...]) * dinv_ref[...]
    h = jnp.maximum(
        jnp.dot(agg, bd1_ref[...], preferred_element_type=jnp.float32)
        + b1t_ref[...], 0.0)
    q = jnp.dot(h, bd2_ref[...], preferred_element_type=jnp.float32)
    qs_ref[...] = q * dinv_ref[...]


def _tc_final(c0_ref, c1_ref, qs_ref, dinv_ref, b2t_ref, out_ref):
    out_ref[...] = ((c0_ref[...] + c1_ref[...] + qs_ref[...])
                    * dinv_ref[...] + b2t_ref[...])


def kernel(x, edge_index, W1, b1, W2, b2):
    n = x.shape[0]
    e = edge_index.shape[1]
    f_in = x.shape[1]
    f_hid = W1.shape[1]
    f_out = W2.shape[1]
    n_pad = _round_up(n, 2048)
    e_pad = _round_up(e, _CHUNK * _INNER * _NW)
    rpt = n_pad // _NS
    m = n_pad * _F // 128
    npl = 128 // _F

    src = edge_index[0].astype(jnp.int32)
    dst = edge_index[1].astype(jnp.int32)
    src_p = jnp.concatenate(
        [src, jnp.zeros((e_pad - e,), jnp.int32)]).reshape(e_pad // _CHUNK, _CHUNK)
    dst_p = jnp.concatenate(
        [dst, jnp.full((e_pad - e,), n, jnp.int32)]).reshape(e_pad // _CHUNK, _CHUNK)
    x8 = jnp.zeros((n_pad, _F), jnp.float32).at[:n, :f_in].set(x)

    zeros8 = jnp.zeros((rpt, _F), jnp.float32)
    ones8 = jnp.ones((_CHUNK, _F), jnp.float32)

    w1p = jnp.zeros((_F, f_hid), jnp.float32).at[:f_in].set(W1)
    w2p = jnp.zeros((f_hid, _F), jnp.float32).at[:, :f_out].set(W2)
    bd1 = jnp.kron(jnp.eye(npl, dtype=jnp.float32), w1p)
    bd2 = jnp.kron(jnp.eye(npl, dtype=jnp.float32), w2p)
    b1t = jnp.tile(b1, npl).reshape(1, npl * f_hid)
    b2t = jnp.tile(jnp.zeros((_F,), jnp.float32).at[:f_out].set(b2),
                   npl).reshape(1, 128)

    def _half(i):
        return pl.BlockSpec((m, 128), lambda g, i=i: (i, 0))

    def _fb(shape):
        return pl.BlockSpec(shape, lambda g: (0, 0))

    deg_parts = _make_deg_kernel(n_pad, e_pad)(dst_p, zeros8, ones8)
    degf = deg_parts.reshape(2 * m, 128)

    dinvf, ysf = pl.pallas_call(
        _tc_norm,
        grid=(1,),
        in_specs=[_half(0), _half(1), _fb((m, 128))],
        out_specs=[_fb((m, 128)), _fb((m, 128))],
        out_shape=[jax.ShapeDtypeStruct((m, 128), jnp.float32),
                   jax.ShapeDtypeStruct((m, 128), jnp.float32)],
    )(degf, degf, x8.reshape(m, 128))

    a_parts = _make_edge_kernel(n_pad, e_pad)(ysf.reshape(n_pad, _F),
                                              src_p, dst_p, zeros8)

    qsf = pl.pallas_call(
        _tc_mid,
        grid=(1,),
        in_specs=[_half(0), _half(1), _fb((m, 128)), _fb((m, 128)),
                  _fb(bd1.shape), _fb(b1t.shape), _fb(bd2.shape)],
        out_specs=_fb((m, 128)),
        out_shape=jax.ShapeDtypeStruct((m, 128), jnp.float32),
    )(a_parts.reshape(2 * m, 128), a_parts.reshape(2 * m, 128), ysf, dinvf,
      bd1, b1t, bd2)

    c_parts = _make_edge_kernel(n_pad, e_pad)(qsf.reshape(n_pad, _F),
                                              src_p, dst_p, zeros8)

    outf = pl.pallas_call(
        _tc_final,
        grid=(1,),
        in_specs=[_half(0), _half(1), _fb((m, 128)), _fb((m, 128)),
                  _fb(b2t.shape)],
        out_specs=_fb((m, 128)),
        out_shape=jax.ShapeDtypeStruct((m, 128), jnp.float32),
    )(c_parts.reshape(2 * m, 128), c_parts.reshape(2 * m, 128), qsf, dinvf,
      b2t)
    return outf.reshape(n_pad, _F)[:n, :f_out]


# speedup vs baseline: 147.1622x; 1.2418x over previous
import functools

import jax
import jax.numpy as jnp
from jax import lax
from jax.experimental import pallas as pl
from jax.experimental.pallas import tpu as pltpu
from jax.experimental.pallas import tpu_sc as plsc

_NC = 2
_NS = 16
_NW = _NC * _NS
_CHUNK = 128
_F = 8
_INNER = 12


def _round_up(v, m):
    return (v + m - 1) // m * m


def _make_deg_kernel(n_pad, e_pad):
    nb = e_pad // (_CHUNK * _NW)
    ng = nb // _INNER
    rpt = n_pad // _NS
    mesh = plsc.VectorSubcoreMesh(core_axis_name="c", subcore_axis_name="s",
                                  num_cores=_NC, num_subcores=_NS)

    @functools.partial(
        pl.kernel,
        out_type=jax.ShapeDtypeStruct((_NC * n_pad, _F), jnp.float32),
        mesh=mesh,
        scratch_types=[
            pltpu.VMEM((2, _INNER, _CHUNK), jnp.int32),
            pltpu.VMEM((_CHUNK, _F), jnp.float32),
            pltpu.VMEM_SHARED((n_pad, _F), jnp.float32),
            pltpu.SemaphoreType.DMA,
            pltpu.SemaphoreType.DMA,
        ],
        compiler_params=pltpu.CompilerParams(use_tc_tiling_on_sc=False),
    )
    def deg_kernel(dst_hbm, zeros_hbm, ones_hbm, out_hbm, didx, obuf, acc,
                   isem, ssem):
        c = lax.axis_index("c")
        s = lax.axis_index("s")
        wid = c * _NS + s
        pltpu.sync_copy(ones_hbm, obuf)
        pltpu.sync_copy(zeros_hbm, acc.at[pl.ds(s * rpt, rpt)])
        plsc.subcore_barrier()
        base = wid * nb
        pltpu.async_copy(dst_hbm.at[pl.ds(base, _INNER)], didx.at[0], isem)

        def group(g, _):
            p = lax.rem(g, 2)

            @pl.when(g >= 1)
            def _drain_prev():
                for j in range(_INNER):
                    pltpu.make_async_copy(
                        obuf, acc.at[didx.at[1 - p, j]], ssem).wait()

            @pl.when(g + 1 < ng)
            def _prefetch():
                pltpu.async_copy(
                    dst_hbm.at[pl.ds(base + (g + 1) * _INNER, _INNER)],
                    didx.at[1 - p], isem)

            pltpu.make_async_copy(
                dst_hbm.at[pl.ds(base, _INNER)], didx.at[p], isem).wait()
            for j in range(_INNER):
                pltpu.async_copy(obuf, acc.at[didx.at[p, j]], ssem, add=True)
            return _

        lax.fori_loop(0, ng, group, None)
        pf = lax.rem(ng - 1, 2)
        for j in range(_INNER):
            pltpu.make_async_copy(obuf, acc.at[didx.at[pf, j]], ssem).wait()
        plsc.subcore_barrier()
        pltpu.sync_copy(acc.at[pl.ds(s * rpt, rpt)],
                        out_hbm.at[pl.ds((c * _NS + s) * rpt, rpt)])

    return deg_kernel


def _make_edge_kernel(n_pad, e_pad):
    nb = e_pad // (_CHUNK * _NW)
    ng = nb // _INNER
    rpt = n_pad // _NS
    mesh = plsc.VectorSubcoreMesh(core_axis_name="c", subcore_axis_name="s",
                                  num_cores=_NC, num_subcores=_NS)

    @functools.partial(
        pl.kernel,
        out_type=jax.ShapeDtypeStruct((_NC * n_pad, _F), jnp.float32),
        mesh=mesh,
        scratch_types=[
            pltpu.VMEM((2, _INNER, _CHUNK), jnp.int32),
            pltpu.VMEM((2, _INNER, _CHUNK), jnp.int32),
            pltpu.VMEM((2, _INNER, _CHUNK, _F), jnp.float32),
            pltpu.VMEM_SHARED((n_pad, _F), jnp.float32),
            pltpu.VMEM_SHARED((n_pad, _F), jnp.float32),
            pltpu.SemaphoreType.DMA,
            pltpu.SemaphoreType.DMA,
            pltpu.SemaphoreType.DMA,
        ],
        compiler_params=pltpu.CompilerParams(use_tc_tiling_on_sc=False),
    )
    def edge_kernel(vals_hbm, src_hbm, dst_hbm, zeros_hbm, out_hbm,
                    sidx, didx, grows, acc, shvals, isem, gsem, ssem):
        c = lax.axis_index("c")
        s = lax.axis_index("s")
        wid = c * _NS + s
        pltpu.sync_copy(vals_hbm.at[pl.ds(s * rpt, rpt)],
                        shvals.at[pl.ds(s * rpt, rpt)])
        pltpu.sync_copy(zeros_hbm, acc.at[pl.ds(s * rpt, rpt)])
        plsc.subcore_barrier()
        base = wid * nb
        pltpu.async_copy(src_hbm.at[pl.ds(base, _INNER)], sidx.at[0], isem)
        pltpu.async_copy(dst_hbm.at[pl.ds(base, _INNER)], didx.at[0], isem)

        def group(g, _):
            p = lax.rem(g, 2)

            @pl.when(g >= 1)
            def _drain_prev():
                for j in range(_INNER):
                    pltpu.make_async_copy(
                        grows.at[1 - p, j], acc.at[didx.at[1 - p, j]],
                        ssem).wait()

            @pl.when(g + 1 < ng)
            def _prefetch():
                off = base + (g + 1) * _INNER
                pltpu.async_copy(src_hbm.at[pl.ds(off, _INNER)],
                                 sidx.at[1 - p], isem)
                pltpu.async_copy(dst_hbm.at[pl.ds(off, _INNER)],
                                 didx.at[1 - p], isem)

            pltpu.make_async_copy(
                src_hbm.at[pl.ds(base, _INNER)], sidx.at[p], isem).wait()
            pltpu.make_async_copy(
                dst_hbm.at[pl.ds(base, _INNER)], didx.at[p], isem).wait()
            for j in range(_INNER):
                pltpu.async_copy(shvals.at[sidx.at[p, j]],
                                 grows.at[p, j], gsem)
            for j in range(_INNER):
                pltpu.make_async_copy(shvals.at[sidx.at[p, j]],
                                      grows.at[p, j], gsem).wait()
                pltpu.async_copy(grows.at[p, j], acc.at[didx.at[p, j]],
                                 ssem, add=True)
            return _

        lax.fori_loop(0, ng, group, None)
        pf = lax.rem(ng - 1, 2)
        for j in range(_INNER):
            pltpu.make_async_copy(
                grows.at[pf, j], acc.at[didx.at[pf, j]], ssem).wait()
        plsc.subcore_barrier()
        pltpu.sync_copy(acc.at[pl.ds(s * rpt, rpt)],
                        out_hbm.at[pl.ds((c * _NS + s) * rpt, rpt)])

    return edge_kernel


def _tc_norm(d0_ref, d1_ref, x8_ref, dinv_ref, ys_ref):
    deg = d0_ref[...] + d1_ref[...] + 1.0
    dinv = lax.rsqrt(deg)
    dinv_ref[...] = dinv
    ys_ref[...] = x8_ref[...] * dinv


def _tc_mid(a0_ref, a1_ref, ys_ref, dinv_ref, bd1_ref, b1t_ref, bd2_ref,
            qs_ref):
    agg = (a0_ref[...] + a1_ref[...] + ys_ref[...]) * dinv_ref[...]
    h = jnp.maximum(
        jnp.dot(agg, bd1_ref[...], preferred_element_type=jnp.float32)
        + b1t_ref[...], 0.0)
    q = jnp.dot(h, bd2_ref[...], preferred_element_type=jnp.float32)
    qs_ref[...] = q * dinv_ref[...]


def _tc_final(c0_ref, c1_ref, qs_ref, dinv_ref, b2t_ref, out_ref):
    out_ref[...] = ((c0_ref[...] + c1_ref[...] + qs_ref[...])
                    * dinv_ref[...] + b2t_ref[...])


def kernel(x, edge_index, W1, b1, W2, b2):
    n = x.shape[0]
    e = edge_index.shape[1]
    f_in = x.shape[1]
    f_hid = W1.shape[1]
    f_out = W2.shape[1]
    n_pad = _round_up(n, 2048)
    e_pad = _round_up(e, _CHUNK * _INNER * _NW)
    rpt = n_pad // _NS
    m = n_pad * _F // 128
    npl = 128 // _F

    src = edge_index[0].astype(jnp.int32)
    dst = edge_index[1].astype(jnp.int32)
    src_p = jnp.concatenate(
        [src, jnp.zeros((e_pad - e,), jnp.int32)]).reshape(e_pad // _CHUNK, _CHUNK)
    dst_p = jnp.concatenate(
        [dst, jnp.full((e_pad - e,), n, jnp.int32)]).reshape(e_pad // _CHUNK, _CHUNK)
    x8 = jnp.zeros((n_pad, _F), jnp.float32).at[:n, :f_in].set(x)

    zeros8 = jnp.zeros((rpt, _F), jnp.float32)
    ones8 = jnp.ones((_CHUNK, _F), jnp.float32)

    w1p = jnp.zeros((_F, f_hid), jnp.float32).at[:f_in].set(W1)
    w2p = jnp.zeros((f_hid, _F), jnp.float32).at[:, :f_out].set(W2)
    bd1 = jnp.kron(jnp.eye(npl, dtype=jnp.float32), w1p)
    bd2 = jnp.kron(jnp.eye(npl, dtype=jnp.float32), w2p)
    b1t = jnp.tile(b1, npl).reshape(1, npl * f_hid)
    b2t = jnp.tile(jnp.zeros((_F,), jnp.float32).at[:f_out].set(b2),
                   npl).reshape(1, 128)

    def _half(i):
        return pl.BlockSpec((m, 128), lambda g, i=i: (i, 0))

    def _fb(shape):
        return pl.BlockSpec(shape, lambda g: (0, 0))

    deg_parts = _make_deg_kernel(n_pad, e_pad)(dst_p, zeros8, ones8)
    degf = deg_parts.reshape(2 * m, 128)

    dinvf, ysf = pl.pallas_call(
        _tc_norm,
        grid=(1,),
        in_specs=[_half(0), _half(1), _fb((m, 128))],
        out_specs=[_fb((m, 128)), _fb((m, 128))],
        out_shape=[jax.ShapeDtypeStruct((m, 128), jnp.float32),
                   jax.ShapeDtypeStruct((m, 128), jnp.float32)],
    )(degf, degf, x8.reshape(m, 128))

    a_parts = _make_edge_kernel(n_pad, e_pad)(ysf.reshape(n_pad, _F),
                                              src_p, dst_p, zeros8)

    qsf = pl.pallas_call(
        _tc_mid,
        grid=(1,),
        in_specs=[_half(0), _half(1), _fb((m, 128)), _fb((m, 128)),
                  _fb(bd1.shape), _fb(b1t.shape), _fb(bd2.shape)],
        out_specs=_fb((m, 128)),
        out_shape=jax.ShapeDtypeStruct((m, 128), jnp.float32),
    )(a_parts.reshape(2 * m, 128), a_parts.reshape(2 * m, 128), ysf, dinvf,
      bd1, b1t, bd2)

    c_parts = _make_edge_kernel(n_pad, e_pad)(qsf.reshape(n_pad, _F),
                                              src_p, dst_p, zeros8)

    outf = pl.pallas_call(
        _tc_final,
        grid=(1,),
        in_specs=[_half(0), _half(1), _fb((m, 128)), _fb((m, 128)),
                  _fb(b2t.shape)],
        out_specs=_fb((m, 128)),
        out_shape=jax.ShapeDtypeStruct((m, 128), jnp.float32),
    )(c_parts.reshape(2 * m, 128), c_parts.reshape(2 * m, 128), qsf, dinvf,
      b2t)
    return outf.reshape(n_pad, _F)[:n, :f_out]
